# speedup vs baseline: 7.3928x; 7.3928x over previous
import jax
import jax.numpy as jnp
from jax.experimental import pallas as pl
from jax.experimental.pallas import tpu as pltpu
from jax.experimental.pallas import tpu_sc as plsc

_WINDOW = 128


def kernel(x, emb_weight):
    B, L = x.shape
    N = B * L
    V, D = emb_weight.shape
    idx = x.reshape(1, N).astype(jnp.int32)

    mesh = plsc.VectorSubcoreMesh(core_axis_name="core", subcore_axis_name="subcore")

    @pl.kernel(
        out_type=jax.ShapeDtypeStruct((N, D), emb_weight.dtype),
        mesh=mesh,
    )
    def run(table_hbm, idx_hbm, out_hbm):
        def body(i_vmem, o_vmem):
            pltpu.sync_copy(table_hbm.at[i_vmem.at[0]], o_vmem)

        pltpu.emit_pipeline(
            body,
            grid=(N // _WINDOW,),
            in_specs=[pl.BlockSpec((1, _WINDOW), index_map=lambda i: (0, i))],
            out_specs=[pl.BlockSpec((_WINDOW, D), index_map=lambda i: (i, 0))],
            core_axis_name=("core", "subcore"),
            dimension_semantics=(pltpu.PARALLEL,),
        )(idx_hbm, out_hbm)

    return run(emb_weight, idx).reshape(B, L, D)


# speedup vs baseline: 9.1990x; 1.2443x over previous
import jax
import jax.numpy as jnp
from jax import lax
from jax.experimental import pallas as pl
from jax.experimental.pallas import tpu as pltpu
from jax.experimental.pallas import tpu_sc as plsc

_CH = 128
_NB = 4


def kernel(x, emb_weight):
    B, L = x.shape
    N = B * L
    V, D = emb_weight.shape
    idx = x.reshape(N).astype(jnp.int32)

    NW = 32
    per_w = N // NW
    steps = per_w // _CH
    mesh = plsc.VectorSubcoreMesh(core_axis_name="core", subcore_axis_name="subcore")

    @pl.kernel(
        out_type=jax.ShapeDtypeStruct((N, D), emb_weight.dtype),
        mesh=mesh,
        scratch_types=(
            [pltpu.VMEM((per_w,), jnp.int32)]
            + [pltpu.VMEM((_CH, D), jnp.float32) for _ in range(_NB)]
            + [pltpu.SemaphoreType.DMA for _ in range(2 * _NB)]
        ),
    )
    def run(table_hbm, idx_hbm, out_hbm, idx_v, *scratch):
        bufs = scratch[:_NB]
        gsem = scratch[_NB:2 * _NB]
        wsem = scratch[2 * _NB:]
        wid = lax.axis_index("subcore") * 2 + lax.axis_index("core")
        base = wid * per_w

        pltpu.sync_copy(idx_hbm.at[pl.ds(base, per_w)], idx_v)

        def gather(g, b):
            return pltpu.make_async_copy(
                table_hbm.at[idx_v.at[pl.ds(g * _CH, _CH)]], bufs[b], gsem[b])

        def write(g, b):
            return pltpu.make_async_copy(
                bufs[b], out_hbm.at[pl.ds(base + g * _CH, _CH)], wsem[b])

        for b in range(_NB):
            gather(b, b).start()

        @pl.loop(0, steps, step=_NB)
        def _(g0):
            for b in range(_NB):
                g = g0 + b
                gather(g, b).wait()
                write(g, b).start()
                write(g, b).wait()

                @pl.when(g + _NB < steps)
                def _():
                    gather(g + _NB, b).start()

    return run(emb_weight, idx).reshape(B, L, D)
